# Initial kernel scaffold; baseline (speedup 1.0000x reference)
import jax
import jax.numpy as jnp
from jax.experimental import pallas as pl


def kernel(h, m, rbf, id_j, W_rbf, W_dense1, W_res0a, W_res0b, W_res1a, W_res1b, W_res2a, W_res2b, scale):
    raise NotImplementedError("write your pallas kernel here")


# speedup vs baseline: 2.0240x; 2.0240x over previous
import functools

import jax
import jax.numpy as jnp
from jax import lax
from jax.experimental import pallas as pl
from jax.experimental.pallas import tpu as pltpu
from jax.experimental.pallas import tpu_sc as plsc

E = 160000
N = 10000
D = 256
DH = 128
R = 16

NC = 2
NS = 16
EP = E // NS
GS = 80
G = EP // GS
WB = 624
WBT = N - NS * WB
ZR = 104

EBLK = 1000
NBLK = 1000


def _edge_fma_body(m_ref, rbf_ref, w_ref, out_ref):
    mlp = jnp.dot(rbf_ref[...], w_ref[...], preferred_element_type=jnp.float32)
    y = m_ref[...] * mlp
    out_ref[0] = y[:, :DH]
    out_ref[1] = y[:, DH:]


def _seg_sum_body(x_hbm, idx_hbm, out_hbm, idx_v, xbuf, zbuf, accum):
    c = lax.axis_index("c")
    s = lax.axis_index("s")

    def zrow(i, carry):
        for k in range(DH // 16):
            zbuf[i, pl.ds(k * 16, 16)] = jnp.zeros((16,), jnp.float32)
        return carry

    lax.fori_loop(0, ZR, zrow, 0)
    for k in range(WB // ZR):
        pltpu.sync_copy(zbuf, accum.at[pl.ds(s * WB + k * ZR, ZR)])

    @pl.when(s == NS - 1)
    def _zero_tail():
        pltpu.sync_copy(zbuf.at[pl.ds(0, WBT)], accum.at[pl.ds(NS * WB, WBT)])

    plsc.subcore_barrier()

    pltpu.sync_copy(idx_hbm.at[s], idx_v)

    def body(g, carry):
        pltpu.sync_copy(x_hbm.at[c, pl.ds(s * EP + g * GS, GS)], xbuf)
        pltpu.sync_copy(xbuf, accum.at[idx_v.at[g]], add=True)
        return carry

    lax.fori_loop(0, G, body, 0)
    plsc.subcore_barrier()

    pltpu.sync_copy(accum.at[pl.ds(s * WB, WB)],
                    out_hbm.at[c, pl.ds(s * WB, WB)])

    @pl.when(s == NS - 1)
    def _write_tail():
        pltpu.sync_copy(accum.at[pl.ds(NS * WB, WBT)],
                        out_hbm.at[c, pl.ds(NS * WB, WBT)])


def _mlp_body(x2_ref, w1_ref, wa0_ref, wb0_ref, wa1_ref, wb1_ref,
              wa2_ref, wb2_ref, out_ref):
    inv_sqrt2 = jnp.float32(0.7071067811865476)
    a = x2_ref[0]
    b = x2_ref[1]
    w1 = w1_ref[...]
    x = jax.nn.silu(
        jnp.dot(a, w1[:DH, :], preferred_element_type=jnp.float32)
        + jnp.dot(b, w1[DH:, :], preferred_element_type=jnp.float32))
    for wa_ref, wb_ref in ((wa0_ref, wb0_ref), (wa1_ref, wb1_ref),
                           (wa2_ref, wb2_ref)):
        y = jax.nn.silu(jnp.dot(x, wa_ref[...],
                                preferred_element_type=jnp.float32))
        y = jax.nn.silu(jnp.dot(y, wb_ref[...],
                                preferred_element_type=jnp.float32))
        x = (x + y) * inv_sqrt2
    out_ref[...] = x


def kernel(h, m, rbf, id_j, W_rbf, W_dense1,
           W_res0a, W_res0b, W_res1a, W_res1b, W_res2a, W_res2b, scale):
    w_rbf_s = W_rbf * scale

    xsplit = pl.pallas_call(
        _edge_fma_body,
        grid=(E // EBLK,),
        in_specs=[
            pl.BlockSpec((EBLK, D), lambda i: (i, 0)),
            pl.BlockSpec((EBLK, R), lambda i: (i, 0)),
            pl.BlockSpec((R, D), lambda i: (0, 0)),
        ],
        out_specs=pl.BlockSpec((NC, EBLK, DH), lambda i: (0, i, 0)),
        out_shape=jax.ShapeDtypeStruct((NC, E, DH), jnp.float32),
    )(m, rbf, w_rbf_s)

    idx3 = id_j.astype(jnp.int32).reshape(NS, G, GS)
    seg = pl.kernel(
        _seg_sum_body,
        out_type=jax.ShapeDtypeStruct((NC, N, DH), jnp.float32),
        mesh=plsc.VectorSubcoreMesh(core_axis_name="c", subcore_axis_name="s"),
        scratch_types=[
            pltpu.VMEM((G, GS), jnp.int32),
            pltpu.VMEM((GS, DH), jnp.float32),
            pltpu.VMEM((ZR, DH), jnp.float32),
            pltpu.VMEM_SHARED((N, DH), jnp.float32),
        ],
    )
    x2 = seg(xsplit, idx3)

    wspec = pl.BlockSpec((D, D), lambda i: (0, 0))
    out = pl.pallas_call(
        _mlp_body,
        grid=(N // NBLK,),
        in_specs=[
            pl.BlockSpec((NC, NBLK, DH), lambda i: (0, i, 0)),
            wspec, wspec, wspec, wspec, wspec, wspec, wspec,
        ],
        out_specs=pl.BlockSpec((NBLK, D), lambda i: (i, 0)),
        out_shape=jax.ShapeDtypeStruct((N, D), jnp.float32),
    )(x2, W_dense1, W_res0a, W_res0b, W_res1a, W_res1b, W_res2a, W_res2b)
    return out


# speedup vs baseline: 2.4765x; 1.2236x over previous
import functools

import jax
import jax.numpy as jnp
from jax import lax
from jax.experimental import pallas as pl
from jax.experimental.pallas import tpu as pltpu
from jax.experimental.pallas import tpu_sc as plsc

E = 160000
N = 10000
D = 256
DH = 128
R = 16

NC = 2
NS = 16
EP = E // NS
GS = 80
G = EP // GS
WB = 624
WBT = N - NS * WB
ZR = 48

EBLK = 1000
NBLK = 1000


def _edge_fma_body(m_ref, rbf_ref, w_ref, out_ref):
    mlp = jnp.dot(rbf_ref[...], w_ref[...], preferred_element_type=jnp.float32)
    y = m_ref[...] * mlp
    out_ref[0] = y[:, :DH]
    out_ref[1] = y[:, DH:]


def _seg_sum_body(x_hbm, idx_hbm, out_hbm, idx_v, xbuf, gsem, ssem, accum):
    c = lax.axis_index("c")
    s = lax.axis_index("s")

    def zrow(i, carry):
        for k in range(DH // 16):
            xbuf[0, i, pl.ds(k * 16, 16)] = jnp.zeros((16,), jnp.float32)
        return carry

    lax.fori_loop(0, ZR, zrow, 0)
    for k in range(WB // ZR):
        pltpu.sync_copy(xbuf.at[0, pl.ds(0, ZR)],
                        accum.at[pl.ds(s * WB + k * ZR, ZR)])

    @pl.when(s == NS - 1)
    def _zero_tail():
        pltpu.sync_copy(xbuf.at[0, pl.ds(0, WBT)],
                        accum.at[pl.ds(NS * WB, WBT)])

    plsc.subcore_barrier()

    pltpu.sync_copy(idx_hbm.at[s], idx_v)

    gd = [None] * G
    sd = [None] * G
    gd[0] = pltpu.async_copy(x_hbm.at[c, pl.ds(s * EP, GS)], xbuf.at[0], gsem)
    for i in range(G):
        if i + 1 < G:
            if i >= 1:
                sd[i - 1].wait()
            gd[i + 1] = pltpu.async_copy(
                x_hbm.at[c, pl.ds(s * EP + (i + 1) * GS, GS)],
                xbuf.at[(i + 1) % 2], gsem)
        gd[i].wait()
        sd[i] = pltpu.async_copy(xbuf.at[i % 2],
                                 accum.at[idx_v.at[i]], ssem, add=True)
    sd[G - 2].wait()
    sd[G - 1].wait()
    plsc.subcore_barrier()

    pltpu.sync_copy(accum.at[pl.ds(s * WB, WB)],
                    out_hbm.at[c, pl.ds(s * WB, WB)])

    @pl.when(s == NS - 1)
    def _write_tail():
        pltpu.sync_copy(accum.at[pl.ds(NS * WB, WBT)],
                        out_hbm.at[c, pl.ds(NS * WB, WBT)])


def _mlp_body(x2_ref, w1_ref, wa0_ref, wb0_ref, wa1_ref, wb1_ref,
              wa2_ref, wb2_ref, out_ref):
    inv_sqrt2 = jnp.float32(0.7071067811865476)
    a = x2_ref[0]
    b = x2_ref[1]
    w1 = w1_ref[...]
    x = jax.nn.silu(
        jnp.dot(a, w1[:DH, :], preferred_element_type=jnp.float32)
        + jnp.dot(b, w1[DH:, :], preferred_element_type=jnp.float32))
    for wa_ref, wb_ref in ((wa0_ref, wb0_ref), (wa1_ref, wb1_ref),
                           (wa2_ref, wb2_ref)):
        y = jax.nn.silu(jnp.dot(x, wa_ref[...],
                                preferred_element_type=jnp.float32))
        y = jax.nn.silu(jnp.dot(y, wb_ref[...],
                                preferred_element_type=jnp.float32))
        x = (x + y) * inv_sqrt2
    out_ref[...] = x


def kernel(h, m, rbf, id_j, W_rbf, W_dense1,
           W_res0a, W_res0b, W_res1a, W_res1b, W_res2a, W_res2b, scale):
    w_rbf_s = W_rbf * scale

    xsplit = pl.pallas_call(
        _edge_fma_body,
        grid=(E // EBLK,),
        in_specs=[
            pl.BlockSpec((EBLK, D), lambda i: (i, 0)),
            pl.BlockSpec((EBLK, R), lambda i: (i, 0)),
            pl.BlockSpec((R, D), lambda i: (0, 0)),
        ],
        out_specs=pl.BlockSpec((NC, EBLK, DH), lambda i: (0, i, 0)),
        out_shape=jax.ShapeDtypeStruct((NC, E, DH), jnp.float32),
    )(m, rbf, w_rbf_s)

    idx3 = id_j.astype(jnp.int32).reshape(NS, G, GS)
    seg = pl.kernel(
        _seg_sum_body,
        out_type=jax.ShapeDtypeStruct((NC, N, DH), jnp.float32),
        mesh=plsc.VectorSubcoreMesh(core_axis_name="c", subcore_axis_name="s"),
        scratch_types=[
            pltpu.VMEM((G, GS), jnp.int32),
            pltpu.VMEM((2, GS, DH), jnp.float32),
            pltpu.SemaphoreType.DMA,
            pltpu.SemaphoreType.DMA,
            pltpu.VMEM_SHARED((N, DH), jnp.float32),
        ],
    )
    x2 = seg(xsplit, idx3)

    wspec = pl.BlockSpec((D, D), lambda i: (0, 0))
    out = pl.pallas_call(
        _mlp_body,
        grid=(N // NBLK,),
        in_specs=[
            pl.BlockSpec((NC, NBLK, DH), lambda i: (0, i, 0)),
            wspec, wspec, wspec, wspec, wspec, wspec, wspec,
        ],
        out_specs=pl.BlockSpec((NBLK, D), lambda i: (i, 0)),
        out_shape=jax.ShapeDtypeStruct((N, D), jnp.float32),
    )(x2, W_dense1, W_res0a, W_res0b, W_res1a, W_res1b, W_res2a, W_res2b)
    return out


# speedup vs baseline: 2.9050x; 1.1730x over previous
import functools

import jax
import jax.numpy as jnp
from jax import lax
from jax.experimental import pallas as pl
from jax.experimental.pallas import tpu as pltpu
from jax.experimental.pallas import tpu_sc as plsc

E = 160000
N = 10000
D = 256
DH = 128
R = 16

NC = 2
NS = 16
EP = E // NS
GS = 80
G = EP // GS
WB = 624
WBT = N - NS * WB
ZR = 48

EBLK = 8000
NBLK = 1000


def _edge_fma_body(m_ref, rbf_ref, w_ref, out_ref):
    mlp = jnp.dot(rbf_ref[...], w_ref[...], preferred_element_type=jnp.float32)
    y = m_ref[...] * mlp
    out_ref[0] = y[:, :DH]
    out_ref[1] = y[:, DH:]


def _seg_sum_body(x_hbm, idx_hbm, out_hbm, idx_v, xbuf, gsem, ssem, accum):
    c = lax.axis_index("c")
    s = lax.axis_index("s")

    for i in range(ZR):
        for k in range(DH // 16):
            xbuf[0, i, pl.ds(k * 16, 16)] = jnp.zeros((16,), jnp.float32)
    for k in range(WB // ZR):
        pltpu.sync_copy(xbuf.at[0, pl.ds(0, ZR)],
                        accum.at[pl.ds(s * WB + k * ZR, ZR)])

    @pl.when(s == NS - 1)
    def _zero_tail():
        pltpu.sync_copy(xbuf.at[0, pl.ds(0, WBT)],
                        accum.at[pl.ds(NS * WB, WBT)])

    plsc.subcore_barrier()

    pltpu.sync_copy(idx_hbm.at[s], idx_v)

    gd = [None] * G
    sd = [None] * G
    gd[0] = pltpu.async_copy(x_hbm.at[c, pl.ds(s * EP, GS)], xbuf.at[0], gsem)
    for i in range(G):
        if i + 1 < G:
            if i >= 1:
                sd[i - 1].wait()
            gd[i + 1] = pltpu.async_copy(
                x_hbm.at[c, pl.ds(s * EP + (i + 1) * GS, GS)],
                xbuf.at[(i + 1) % 2], gsem)
        gd[i].wait()
        sd[i] = pltpu.async_copy(xbuf.at[i % 2],
                                 accum.at[idx_v.at[i]], ssem, add=True)
    sd[G - 2].wait()
    sd[G - 1].wait()
    plsc.subcore_barrier()

    pltpu.sync_copy(accum.at[pl.ds(s * WB, WB)],
                    out_hbm.at[c, pl.ds(s * WB, WB)])

    @pl.when(s == NS - 1)
    def _write_tail():
        pltpu.sync_copy(accum.at[pl.ds(NS * WB, WBT)],
                        out_hbm.at[c, pl.ds(NS * WB, WBT)])


def _mlp_body(x2_ref, w1_ref, wa0_ref, wb0_ref, wa1_ref, wb1_ref,
              wa2_ref, wb2_ref, out_ref):
    inv_sqrt2 = jnp.float32(0.7071067811865476)
    a = x2_ref[0]
    b = x2_ref[1]
    w1 = w1_ref[...]
    x = jax.nn.silu(
        jnp.dot(a, w1[:DH, :], preferred_element_type=jnp.float32)
        + jnp.dot(b, w1[DH:, :], preferred_element_type=jnp.float32))
    for wa_ref, wb_ref in ((wa0_ref, wb0_ref), (wa1_ref, wb1_ref),
                           (wa2_ref, wb2_ref)):
        y = jax.nn.silu(jnp.dot(x, wa_ref[...],
                                preferred_element_type=jnp.float32))
        y = jax.nn.silu(jnp.dot(y, wb_ref[...],
                                preferred_element_type=jnp.float32))
        x = (x + y) * inv_sqrt2
    out_ref[...] = x


def kernel(h, m, rbf, id_j, W_rbf, W_dense1,
           W_res0a, W_res0b, W_res1a, W_res1b, W_res2a, W_res2b, scale):
    w_rbf_s = W_rbf * scale

    xsplit = pl.pallas_call(
        _edge_fma_body,
        grid=(E // EBLK,),
        in_specs=[
            pl.BlockSpec((EBLK, D), lambda i: (i, 0)),
            pl.BlockSpec((EBLK, R), lambda i: (i, 0)),
            pl.BlockSpec((R, D), lambda i: (0, 0)),
        ],
        out_specs=pl.BlockSpec((NC, EBLK, DH), lambda i: (0, i, 0)),
        out_shape=jax.ShapeDtypeStruct((NC, E, DH), jnp.float32),
    )(m, rbf, w_rbf_s)

    idx3 = id_j.astype(jnp.int32).reshape(NS, G, GS)
    seg = pl.kernel(
        _seg_sum_body,
        out_type=jax.ShapeDtypeStruct((NC, N, DH), jnp.float32),
        mesh=plsc.VectorSubcoreMesh(core_axis_name="c", subcore_axis_name="s"),
        scratch_types=[
            pltpu.VMEM((G, GS), jnp.int32),
            pltpu.VMEM((2, GS, DH), jnp.float32),
            pltpu.SemaphoreType.DMA,
            pltpu.SemaphoreType.DMA,
            pltpu.VMEM_SHARED((N, DH), jnp.float32),
        ],
    )
    x2 = seg(xsplit, idx3)

    wspec = pl.BlockSpec((D, D), lambda i: (0, 0))
    out = pl.pallas_call(
        _mlp_body,
        grid=(N // NBLK,),
        in_specs=[
            pl.BlockSpec((NC, NBLK, DH), lambda i: (0, i, 0)),
            wspec, wspec, wspec, wspec, wspec, wspec, wspec,
        ],
        out_specs=pl.BlockSpec((NBLK, D), lambda i: (i, 0)),
        out_shape=jax.ShapeDtypeStruct((N, D), jnp.float32),
    )(x2, W_dense1, W_res0a, W_res0b, W_res1a, W_res1b, W_res2a, W_res2b)
    return out
